# baseline (speedup 1.0000x reference)
import functools

import jax
import jax.numpy as jnp
from jax import lax
from jax.experimental import pallas as pl
from jax.experimental.pallas import tpu as pltpu
from jax.experimental.pallas import tpu_sc as plsc

_N = 10000
_NPAD = 10240
_E = 320000
_EPW = 79 * 128
_EPAD = 32 * _EPW
_AGGPAD = _NPAD + 128

def _sortable(f):
    f = jnp.where(f == 0.0, 0.0, f)
    b = lax.bitcast_convert_type(f, jnp.int32)
    return jnp.where(b >= 0, b, (~b) ^ (-2147483648))


_JBLK = 1024


def _rank_body(kcol_ref, krow_ref, rank_ref):
    j = pl.program_id(0)
    sj = _sortable(-kcol_ref[...])
    jg = j * _JBLK + lax.broadcasted_iota(jnp.int32, (_JBLK, 1), 0)

    def body(kt, acc):
        sk = _sortable(-krow_ref[kt])
        kg = kt * 128 + lax.broadcasted_iota(jnp.int32, (1, 128), 1)
        lt = sk < sj
        eq = (sk == sj) & (kg < jg)
        return acc + jnp.sum((lt | eq).astype(jnp.int32), axis=1, keepdims=True)

    acc = lax.fori_loop(0, _NPAD // 128, body,
                        jnp.zeros((_JBLK, 1), jnp.int32))
    rank_ref[...] = acc


def _rank_call(keys_col, keys_row):
    return pl.pallas_call(
        _rank_body,
        grid=(_NPAD // _JBLK,),
        in_specs=[
            pl.BlockSpec((_JBLK, 1), lambda j: (j, 0)),
            pl.BlockSpec((_NPAD // 128, 1, 128), lambda j: (0, 0, 0)),
        ],
        out_specs=pl.BlockSpec((_JBLK, 1), lambda j: (j, 0)),
        out_shape=jax.ShapeDtypeStruct((_NPAD, 1), jnp.int32),
    )(keys_col, keys_row)


@functools.cache
def _sc_scatter_fn():
    mesh = plsc.VectorSubcoreMesh(core_axis_name="c", subcore_axis_name="s")

    @functools.partial(
        pl.kernel,
        out_type=(
            jax.ShapeDtypeStruct((_NPAD,), jnp.float32),
            jax.ShapeDtypeStruct((2, _AGGPAD), jnp.float32),
            jax.ShapeDtypeStruct((2, _AGGPAD), jnp.float32),
        ),
        mesh=mesh,
        scratch_types=[
            pltpu.VMEM((5, 128), jnp.int32),
            pltpu.VMEM((5, 128), jnp.float32),
            pltpu.VMEM((79, 128), jnp.int32),
            pltpu.VMEM((79, 128), jnp.int32),
            pltpu.VMEM((79, 128), jnp.float32),
            pltpu.VMEM((128,), jnp.float32),
            pltpu.VMEM_SHARED((_NPAD,), jnp.float32),
            pltpu.VMEM_SHARED((_AGGPAD,), jnp.float32),
            pltpu.VMEM_SHARED((_AGGPAD,), jnp.float32),
        ],
    )
    def _sc_scatter(rank_in, x0_in, src_in, dst_in, zeros_in,
                    h_out, agg_out, cnt_out,
                    rank_v, x0_v, src_v, dst_v, hv_v, ones_v,
                    h_sh, agg_sh, cnt_sh):
        _sc_body(rank_in, x0_in, src_in, dst_in, zeros_in,
                 h_out, agg_out, cnt_out,
                 rank_v, x0_v, src_v, dst_v, hv_v, ones_v,
                 h_sh, agg_sh, cnt_sh)

    return _sc_scatter


def _sc_body(rank_in, x0_in, src_in, dst_in, zeros_in,
             h_out, agg_out, cnt_out,
             rank_v, x0_v, src_v, dst_v, hv_v, ones_v,
             h_sh, agg_sh, cnt_sh):
    c = lax.axis_index("c")
    s = lax.axis_index("s")
    wid = c * 16 + s

    pltpu.sync_copy(rank_in.at[s], rank_v)
    pltpu.sync_copy(x0_in.at[s], x0_v)
    for r in range(5):
        pltpu.sync_copy(x0_v.at[r], h_sh.at[rank_v.at[r]])

    @pl.when(s == 0)
    def _zero():
        pltpu.sync_copy(zeros_in, agg_sh)
        pltpu.sync_copy(zeros_in, cnt_sh)

    for i in range(8):
        ones_v[pl.ds(i * 16, 16)] = jnp.full((16,), 1.0, jnp.float32)

    plsc.subcore_barrier()

    pltpu.sync_copy(src_in.at[wid], src_v)
    pltpu.sync_copy(dst_in.at[wid], dst_v)

    def gbody(r, carry):
        pltpu.sync_copy(h_sh.at[src_v.at[r]], hv_v.at[r])
        return carry

    lax.fori_loop(0, 79, gbody, 0)

    def sbody(r, carry):
        pltpu.sync_copy(hv_v.at[r], agg_sh.at[dst_v.at[r]], add=True)
        pltpu.sync_copy(ones_v, cnt_sh.at[dst_v.at[r]], add=True)
        return carry

    lax.fori_loop(0, 79, sbody, 0)

    plsc.subcore_barrier()

    @pl.when(s == 0)
    def _emit():
        pltpu.sync_copy(agg_sh, agg_out.at[c])
        pltpu.sync_copy(cnt_sh, cnt_out.at[c])

    @pl.when((s == 0) & (c == 0))
    def _emit_h():
        pltpu.sync_copy(h_sh, h_out)


def _dense_body(cols_ref, wl_ref, bl_ref, wr_ref, wfc1_ref, bfc1_ref,
                wfc2_ref, bfc2_ref, o_ref):
    cols = cols_ref[...]
    h = cols[:, 0:1]
    agg = cols[:, 1:2] + cols[:, 2:3]
    cnt = cols[:, 3:4] + cols[:, 4:5]
    m = agg / jnp.maximum(cnt, 1.0)
    out = m * wl_ref[...] + bl_ref[...] + h * wr_ref[...]
    nrm = jnp.sqrt(jnp.sum(out * out, axis=1, keepdims=True))
    out = out / jnp.maximum(nrm, 1e-12)
    z = jnp.dot(out, wfc1_ref[...], preferred_element_type=jnp.float32)
    z = jnp.maximum(z + bfc1_ref[...], 0.0)
    y = jnp.dot(z, wfc2_ref[...], preferred_element_type=jnp.float32)
    y = y + bfc2_ref[...]
    mx = jnp.max(y, axis=0, keepdims=True)
    lse = mx + jnp.log(jnp.sum(jnp.exp(y - mx), axis=0, keepdims=True))
    o_ref[...] = y - lse


def _dense_call(cols, W_l, b_l, W_r, W_fc1, b_fc1, W_fc2, b_fc2):
    return pl.pallas_call(
        _dense_body,
        out_shape=jax.ShapeDtypeStruct((_N, 6), jnp.float32),
    )(cols, W_l, b_l, W_r, W_fc1, b_fc1, W_fc2, b_fc2)


def kernel(x, edge_index, batch, W_l, b_l, W_r, W_fc1, b_fc1, W_fc2, b_fc2):
    npad = _NPAD - _N
    key = x[:, x.shape[1] - 1]
    kp = jnp.concatenate([key, jnp.full((npad,), -jnp.inf, jnp.float32)])
    order = jnp.argsort(-kp)
    rank = jnp.zeros((_NPAD,), jnp.int32).at[order].set(
        jnp.arange(_NPAD, dtype=jnp.int32))[:, None]

    x0p = jnp.concatenate([x[:, 0], jnp.zeros((npad,), jnp.float32)])
    srcp = jnp.concatenate(
        [edge_index[0], jnp.zeros((_EPAD - _E,), jnp.int32)])
    dead = _NPAD + (jnp.arange(_EPAD - _E, dtype=jnp.int32) % 128)
    dstp = jnp.concatenate([edge_index[1], dead])

    h_srt = jnp.zeros((_NPAD,), jnp.float32).at[rank[:, 0]].set(x0p)
    hv = h_srt[srcp]
    aggf = jnp.zeros((_AGGPAD,), jnp.float32).at[dstp].add(hv)
    cntf = jnp.zeros((_AGGPAD,), jnp.float32).at[dstp].add(1.0)
    agg = jnp.stack([aggf, jnp.zeros_like(aggf)])
    cnt = jnp.stack([cntf, jnp.zeros_like(cntf)])

    cols = jnp.stack(
        [h_srt[:_N], agg[0, :_N], agg[1, :_N], cnt[0, :_N], cnt[1, :_N]],
        axis=1)
    return _dense_call(cols, W_l, b_l[None, :], W_r, W_fc1, b_fc1[None, :],
                       W_fc2, b_fc2[None, :])


# speedup vs baseline: 7.7325x; 7.7325x over previous
import functools

import jax
import jax.numpy as jnp
from jax import lax
from jax.experimental import pallas as pl
from jax.experimental.pallas import tpu as pltpu
from jax.experimental.pallas import tpu_sc as plsc

_N = 10000
_NPAD = 10240
_E = 320000
_EPW = 79 * 128
_EPAD = 32 * _EPW
_AGGPAD = _NPAD + 128

def _sortable(f):
    f = jnp.where(f == 0.0, 0.0, f)
    b = lax.bitcast_convert_type(f, jnp.int32)
    return jnp.where(b >= 0, b, (~b) ^ (-2147483648))


_JBLK = 1024


def _rank_body(kcol_ref, krow_ref, rank_ref):
    j = pl.program_id(0)
    sj = _sortable(-kcol_ref[...])
    jg = j * _JBLK + lax.broadcasted_iota(jnp.int32, (_JBLK, 1), 0)
    ones = jnp.ones((128, 1), jnp.float32)
    nt = _JBLK // 128

    def red(cond, acc):
        return acc + jnp.dot(jnp.where(cond, 1.0, 0.0), ones,
                             preferred_element_type=jnp.float32)

    def left(kt, acc):
        return red(_sortable(-krow_ref[kt]) <= sj, acc)

    def diag(kt, acc):
        sk = _sortable(-krow_ref[kt])
        kg = kt * 128 + lax.broadcasted_iota(jnp.int32, (1, 128), 1)
        return red((sk < sj) | ((sk == sj) & (kg < jg)), acc)

    def right(kt, acc):
        return red(_sortable(-krow_ref[kt]) < sj, acc)

    acc = jnp.zeros((_JBLK, 1), jnp.float32)
    acc = lax.fori_loop(0, j * nt, left, acc)
    acc = lax.fori_loop(j * nt, j * nt + nt, diag, acc)
    acc = lax.fori_loop(j * nt + nt, _NPAD // 128, right, acc)
    rank_ref[...] = acc.astype(jnp.int32)


def _rank_call(keys_col, keys_row):
    return pl.pallas_call(
        _rank_body,
        grid=(_NPAD // _JBLK,),
        in_specs=[
            pl.BlockSpec((_JBLK, 1), lambda j: (j, 0)),
            pl.BlockSpec((_NPAD // 128, 1, 128), lambda j: (0, 0, 0)),
        ],
        out_specs=pl.BlockSpec((_JBLK, 1), lambda j: (j, 0)),
        out_shape=jax.ShapeDtypeStruct((_NPAD, 1), jnp.int32),
    )(keys_col, keys_row)


@functools.cache
def _sc_scatter_fn():
    mesh = plsc.VectorSubcoreMesh(core_axis_name="c", subcore_axis_name="s")

    @functools.partial(
        pl.kernel,
        out_type=(
            jax.ShapeDtypeStruct((_NPAD,), jnp.float32),
            jax.ShapeDtypeStruct((2, _AGGPAD), jnp.float32),
            jax.ShapeDtypeStruct((2, _AGGPAD), jnp.float32),
        ),
        mesh=mesh,
        scratch_types=[
            pltpu.VMEM((5, 128), jnp.int32),
            pltpu.VMEM((5, 128), jnp.float32),
            pltpu.VMEM((79, 128), jnp.int32),
            pltpu.VMEM((79, 128), jnp.int32),
            pltpu.VMEM((79, 128), jnp.float32),
            pltpu.VMEM((128,), jnp.float32),
            pltpu.VMEM_SHARED((_NPAD,), jnp.float32),
            pltpu.VMEM_SHARED((_AGGPAD,), jnp.float32),
            pltpu.VMEM_SHARED((_AGGPAD,), jnp.float32),
        ],
    )
    def _sc_scatter(rank_in, x0_in, src_in, dst_in, zeros_in,
                    h_out, agg_out, cnt_out,
                    rank_v, x0_v, src_v, dst_v, hv_v, ones_v,
                    h_sh, agg_sh, cnt_sh):
        _sc_body(rank_in, x0_in, src_in, dst_in, zeros_in,
                 h_out, agg_out, cnt_out,
                 rank_v, x0_v, src_v, dst_v, hv_v, ones_v,
                 h_sh, agg_sh, cnt_sh)

    return _sc_scatter


def _sc_body(rank_in, x0_in, src_in, dst_in, zeros_in,
             h_out, agg_out, cnt_out,
             rank_v, x0_v, src_v, dst_v, hv_v, ones_v,
             h_sh, agg_sh, cnt_sh):
    c = lax.axis_index("c")
    s = lax.axis_index("s")
    wid = c * 16 + s

    pltpu.sync_copy(rank_in.at[s], rank_v)
    pltpu.sync_copy(x0_in.at[s], x0_v)
    for r in range(5):
        pltpu.sync_copy(x0_v.at[r], h_sh.at[rank_v.at[r]])

    @pl.when(s == 0)
    def _zero():
        pltpu.sync_copy(zeros_in, agg_sh)
        pltpu.sync_copy(zeros_in, cnt_sh)

    for i in range(8):
        ones_v[pl.ds(i * 16, 16)] = jnp.full((16,), 1.0, jnp.float32)

    plsc.subcore_barrier()

    pltpu.sync_copy(src_in.at[wid], src_v)
    pltpu.sync_copy(dst_in.at[wid], dst_v)

    def gbody(r, carry):
        pltpu.sync_copy(h_sh.at[src_v.at[r]], hv_v.at[r])
        return carry

    lax.fori_loop(0, 79, gbody, 0)

    def sbody(r, carry):
        pltpu.sync_copy(hv_v.at[r], agg_sh.at[dst_v.at[r]], add=True)
        pltpu.sync_copy(ones_v, cnt_sh.at[dst_v.at[r]], add=True)
        return carry

    lax.fori_loop(0, 79, sbody, 0)

    plsc.subcore_barrier()

    @pl.when(s == 0)
    def _emit():
        pltpu.sync_copy(agg_sh, agg_out.at[c])
        pltpu.sync_copy(cnt_sh, cnt_out.at[c])

    @pl.when((s == 0) & (c == 0))
    def _emit_h():
        pltpu.sync_copy(h_sh, h_out)


def _dense_body(cols_ref, wl_ref, bl_ref, wr_ref, wfc1_ref, bfc1_ref,
                wfc2_ref, bfc2_ref, o_ref):
    cols = cols_ref[...]
    h = cols[:, 0:1]
    agg = cols[:, 1:2] + cols[:, 2:3]
    cnt = cols[:, 3:4] + cols[:, 4:5]
    m = agg / jnp.maximum(cnt, 1.0)
    out = m * wl_ref[...] + bl_ref[...] + h * wr_ref[...]
    nrm = jnp.sqrt(jnp.sum(out * out, axis=1, keepdims=True))
    out = out / jnp.maximum(nrm, 1e-12)
    z = jnp.dot(out, wfc1_ref[...], preferred_element_type=jnp.float32)
    z = jnp.maximum(z + bfc1_ref[...], 0.0)
    y = jnp.dot(z, wfc2_ref[...], preferred_element_type=jnp.float32)
    y = y + bfc2_ref[...]
    mx = jnp.max(y, axis=0, keepdims=True)
    lse = mx + jnp.log(jnp.sum(jnp.exp(y - mx), axis=0, keepdims=True))
    o_ref[...] = y - lse


def _dense_call(cols, W_l, b_l, W_r, W_fc1, b_fc1, W_fc2, b_fc2):
    return pl.pallas_call(
        _dense_body,
        out_shape=jax.ShapeDtypeStruct((_N, 6), jnp.float32),
    )(cols, W_l, b_l, W_r, W_fc1, b_fc1, W_fc2, b_fc2)


def kernel(x, edge_index, batch, W_l, b_l, W_r, W_fc1, b_fc1, W_fc2, b_fc2):
    npad = _NPAD - _N
    key = x[:, x.shape[1] - 1]
    kp = jnp.concatenate([key, jnp.full((npad,), -jnp.inf, jnp.float32)])
    rank = _rank_call(kp[:, None], kp.reshape(_NPAD // 128, 1, 128))

    x0p = jnp.concatenate([x[:, 0], jnp.zeros((npad,), jnp.float32)])
    srcp = jnp.concatenate(
        [edge_index[0], jnp.zeros((_EPAD - _E,), jnp.int32)])
    dead = _NPAD + (jnp.arange(_EPAD - _E, dtype=jnp.int32) % 128)
    dstp = jnp.concatenate([edge_index[1], dead])

    h_srt, agg, cnt = _sc_scatter_fn()(
        rank.reshape(16, 5, 128),
        x0p.reshape(16, 5, 128),
        srcp.reshape(32, 79, 128),
        dstp.reshape(32, 79, 128),
        jnp.zeros((_AGGPAD,), jnp.float32),
    )

    cols = jnp.stack(
        [h_srt[:_N], agg[0, :_N], agg[1, :_N], cnt[0, :_N], cnt[1, :_N]],
        axis=1)
    return _dense_call(cols, W_l, b_l[None, :], W_r, W_fc1, b_fc1[None, :],
                       W_fc2, b_fc2[None, :])


# speedup vs baseline: 24.0624x; 3.1119x over previous
import functools

import jax
import jax.numpy as jnp
from jax import lax
from jax.experimental import pallas as pl
from jax.experimental.pallas import tpu as pltpu
from jax.experimental.pallas import tpu_sc as plsc

_N = 10000
_NPAD = 10240
_E = 320000
_EPW = 79 * 128
_EPAD = 32 * _EPW
_AGGPAD = _NPAD + 128

def _sortable(f):
    f = jnp.where(f == 0.0, 0.0, f)
    b = lax.bitcast_convert_type(f, jnp.int32)
    return jnp.where(b >= 0, b, (~b) ^ (-2147483648))


_JBLK = 2048


def _rank_body(kcol_ref, krow_ref, rank_ref):
    j = pl.program_id(0)
    sj = _sortable(-kcol_ref[...])
    sjb = jnp.broadcast_to(sj, (_JBLK, 128))
    jg = j * _JBLK + lax.broadcasted_iota(jnp.int32, (_JBLK, 1), 0)
    ones = jnp.ones((128, 1), jnp.float32)
    nss = _NPAD // _JBLK
    tpb = _JBLK // 128

    def super_le(ss, acc):
        part = jnp.zeros((_JBLK, 128), jnp.float32)
        for t in range(tpb):
            sk = _sortable(-krow_ref[ss * tpb + t])
            part = part + jnp.where(sk <= sjb, 1.0, 0.0)
        return acc + jnp.dot(part, ones, preferred_element_type=jnp.float32)

    def super_lt(ss, acc):
        part = jnp.zeros((_JBLK, 128), jnp.float32)
        for t in range(tpb):
            sk = _sortable(-krow_ref[ss * tpb + t])
            part = part + jnp.where(sk < sjb, 1.0, 0.0)
        return acc + jnp.dot(part, ones, preferred_element_type=jnp.float32)

    def super_diag(ss, acc):
        part = jnp.zeros((_JBLK, 128), jnp.float32)
        for t in range(tpb):
            kt = ss * tpb + t
            sk = _sortable(-krow_ref[kt])
            kg = kt * 128 + lax.broadcasted_iota(jnp.int32, (1, 128), 1)
            cond = (sk < sjb) | ((sk == sjb) & (kg < jg))
            part = part + jnp.where(cond, 1.0, 0.0)
        return acc + jnp.dot(part, ones, preferred_element_type=jnp.float32)

    acc = jnp.zeros((_JBLK, 1), jnp.float32)
    acc = lax.fori_loop(0, j, super_le, acc)
    acc = super_diag(j, acc)
    acc = lax.fori_loop(j + 1, nss, super_lt, acc)
    rank_ref[...] = acc.astype(jnp.int32)


def _rank_call(keys_col, keys_row):
    return pl.pallas_call(
        _rank_body,
        grid=(_NPAD // _JBLK,),
        in_specs=[
            pl.BlockSpec((_JBLK, 1), lambda j: (j, 0)),
            pl.BlockSpec((_NPAD // 128, 1, 128), lambda j: (0, 0, 0)),
        ],
        out_specs=pl.BlockSpec((_JBLK, 1), lambda j: (j, 0)),
        out_shape=jax.ShapeDtypeStruct((_NPAD, 1), jnp.int32),
    )(keys_col, keys_row)


@functools.cache
def _sc_scatter_fn():
    mesh = plsc.VectorSubcoreMesh(core_axis_name="c", subcore_axis_name="s")

    @functools.partial(
        pl.kernel,
        out_type=(
            jax.ShapeDtypeStruct((_NPAD,), jnp.float32),
            jax.ShapeDtypeStruct((2, _AGGPAD), jnp.float32),
            jax.ShapeDtypeStruct((2, _AGGPAD), jnp.float32),
        ),
        mesh=mesh,
        scratch_types=[
            pltpu.VMEM((5, 128), jnp.int32),
            pltpu.VMEM((5, 128), jnp.float32),
            pltpu.VMEM((79, 128), jnp.int32),
            pltpu.VMEM((79, 128), jnp.int32),
            pltpu.VMEM((79, 128), jnp.float32),
            pltpu.VMEM((128,), jnp.float32),
            pltpu.VMEM_SHARED((_NPAD,), jnp.float32),
            pltpu.VMEM_SHARED((_AGGPAD,), jnp.float32),
            pltpu.VMEM_SHARED((_AGGPAD,), jnp.float32),
            pltpu.SemaphoreType.DMA,
        ],
    )
    def _sc_scatter(rank_in, x0_in, src_in, dst_in, zeros_in,
                    h_out, agg_out, cnt_out,
                    rank_v, x0_v, src_v, dst_v, hv_v, ones_v,
                    h_sh, agg_sh, cnt_sh, sem):
        _sc_body(rank_in, x0_in, src_in, dst_in, zeros_in,
                 h_out, agg_out, cnt_out,
                 rank_v, x0_v, src_v, dst_v, hv_v, ones_v,
                 h_sh, agg_sh, cnt_sh, sem)

    return _sc_scatter


def _sc_body(rank_in, x0_in, src_in, dst_in, zeros_in,
             h_out, agg_out, cnt_out,
             rank_v, x0_v, src_v, dst_v, hv_v, ones_v,
             h_sh, agg_sh, cnt_sh, sem):
    c = lax.axis_index("c")
    s = lax.axis_index("s")
    wid = c * 16 + s

    edge_in_a = pltpu.async_copy(src_in.at[wid], src_v, sem)
    edge_in_b = pltpu.async_copy(dst_in.at[wid], dst_v, sem)

    pltpu.sync_copy(rank_in.at[s], rank_v)
    pltpu.sync_copy(x0_in.at[s], x0_v)
    for r in range(5):
        pltpu.async_copy(x0_v.at[r], h_sh.at[rank_v.at[r]], sem)

    @pl.when(s == 0)
    def _zero():
        pltpu.sync_copy(zeros_in, agg_sh)
        pltpu.sync_copy(zeros_in, cnt_sh)

    for i in range(8):
        ones_v[pl.ds(i * 16, 16)] = jnp.full((16,), 1.0, jnp.float32)

    for r in range(5):
        pltpu.make_async_copy(x0_v.at[r], h_sh.at[rank_v.at[r]], sem).wait()
    edge_in_a.wait()
    edge_in_b.wait()

    plsc.subcore_barrier()

    def gfire(r, carry):
        pltpu.async_copy(h_sh.at[src_v.at[r]], hv_v.at[r], sem)
        pltpu.async_copy(ones_v, cnt_sh.at[dst_v.at[r]], sem, add=True)
        return carry

    lax.fori_loop(0, 79, gfire, 0)

    def gdrain(r, carry):
        pltpu.make_async_copy(h_sh.at[src_v.at[r]], hv_v.at[r], sem).wait()
        pltpu.make_async_copy(ones_v, cnt_sh.at[dst_v.at[r]], sem).wait()
        return carry

    lax.fori_loop(0, 79, gdrain, 0)

    def sfire(r, carry):
        pltpu.async_copy(hv_v.at[r], agg_sh.at[dst_v.at[r]], sem, add=True)
        return carry

    lax.fori_loop(0, 79, sfire, 0)

    def sdrain(r, carry):
        pltpu.make_async_copy(hv_v.at[r], agg_sh.at[dst_v.at[r]], sem).wait()
        return carry

    lax.fori_loop(0, 79, sdrain, 0)

    plsc.subcore_barrier()

    @pl.when(s == 0)
    def _emit():
        pltpu.sync_copy(agg_sh, agg_out.at[c])
        pltpu.sync_copy(cnt_sh, cnt_out.at[c])

    @pl.when((s == 0) & (c == 0))
    def _emit_h():
        pltpu.sync_copy(h_sh, h_out)


def _dense_body(h_ref, agg_ref, cnt_ref, wlt_ref, blt_ref, wrt_ref,
                wfc1t_ref, bfc1t_ref, wfc2t_ref, bfc2t_ref, o_ref):
    h = h_ref[...]
    agg = agg_ref[0:1, :_NPAD] + agg_ref[1:2, :_NPAD]
    cnt = cnt_ref[0:1, :_NPAD] + cnt_ref[1:2, :_NPAD]
    m = agg / jnp.maximum(cnt, 1.0)
    wl = wlt_ref[...]
    bl = blt_ref[...]
    wr = wrt_ref[...]
    u = jnp.dot(wfc1t_ref[...], wl, preferred_element_type=jnp.float32)
    v = jnp.dot(wfc1t_ref[...], wr, preferred_element_type=jnp.float32)
    w = jnp.dot(wfc1t_ref[...], bl, preferred_element_type=jnp.float32)
    qll = jnp.sum(wl * wl)
    qlr = jnp.sum(wl * wr)
    qrr = jnp.sum(wr * wr)
    qlb = jnp.sum(wl * bl)
    qrb = jnp.sum(wr * bl)
    qbb = jnp.sum(bl * bl)
    n2 = (qll * m * m + qrr * h * h + qbb
          + 2.0 * (qlr * m * h + qlb * m + qrb * h))
    nrm = jnp.maximum(jnp.sqrt(jnp.maximum(n2, 0.0)), 1e-12)
    z = (u * m + v * h + w) / nrm + bfc1t_ref[...]
    z = jnp.maximum(z, 0.0)
    y = jnp.dot(wfc2t_ref[...], z, preferred_element_type=jnp.float32)
    y = (y + bfc2t_ref[...])[:, :_N]
    mx = jnp.max(y, axis=1, keepdims=True)
    lse = mx + jnp.log(jnp.sum(jnp.exp(y - mx), axis=1, keepdims=True))
    o_ref[...] = y - lse


def _dense_call(h_row, agg, cnt, W_l, b_l, W_r, W_fc1, b_fc1, W_fc2, b_fc2):
    return pl.pallas_call(
        _dense_body,
        out_shape=jax.ShapeDtypeStruct((6, _N), jnp.float32),
    )(h_row, agg, cnt,
      W_l.reshape(-1, 1), b_l.reshape(-1, 1), W_r.reshape(-1, 1),
      W_fc1.T, b_fc1.reshape(-1, 1), W_fc2.T, b_fc2.reshape(-1, 1))


def kernel(x, edge_index, batch, W_l, b_l, W_r, W_fc1, b_fc1, W_fc2, b_fc2):
    npad = _NPAD - _N
    key = x[:, x.shape[1] - 1]
    kp = jnp.concatenate([key, jnp.full((npad,), -jnp.inf, jnp.float32)])
    rank = _rank_call(kp[:, None], kp.reshape(_NPAD // 128, 1, 128))

    x0p = jnp.concatenate([x[:, 0], jnp.zeros((npad,), jnp.float32)])
    srcp = jnp.concatenate(
        [edge_index[0], jnp.zeros((_EPAD - _E,), jnp.int32)])
    dead = _NPAD + (jnp.arange(_EPAD - _E, dtype=jnp.int32) % 128)
    dstp = jnp.concatenate([edge_index[1], dead])

    h_srt, agg, cnt = _sc_scatter_fn()(
        rank.reshape(16, 5, 128),
        x0p.reshape(16, 5, 128),
        srcp.reshape(32, 79, 128),
        dstp.reshape(32, 79, 128),
        jnp.zeros((_AGGPAD,), jnp.float32),
    )

    out_t = _dense_call(h_srt[None, :], agg, cnt, W_l, b_l, W_r,
                        W_fc1, b_fc1, W_fc2, b_fc2)
    return out_t.T
